# baseline (speedup 1.0000x reference)
import functools

import jax
import jax.numpy as jnp
from jax import lax
from jax.experimental import pallas as pl
from jax.experimental.pallas import tpu as pltpu
from jax.experimental.pallas import tpu_sc as plsc

_N = 10000
_E = 320000
_D = 128
_G = 128
_O = 10
_EPS = 1e-5

_NC = 2
_NS = 16
_NW = _NC * _NS
_EPT = _E // _NW
_CH = 100
_NCH = _EPT // _CH
_NH = 2
_HCH = _NCH // _NH
_NP = 10240
_RPT = _NP // _NS


def _mesh():
    return plsc.VectorSubcoreMesh(core_axis_name="c", subcore_axis_name="s")


def _sc_degree(dst):
    @functools.partial(
        pl.kernel,
        mesh=_mesh(),
        out_type=jax.ShapeDtypeStruct((_NC, _NS * 80, _D), jnp.float32),
        scratch_types=[
            pltpu.VMEM((_NH, _HCH, _CH), jnp.int32),
            pltpu.VMEM((_CH, 16), jnp.float32),
            pltpu.VMEM((16, 16), jnp.float32),
            pltpu.VMEM((64, 16), jnp.float32),
            pltpu.VMEM((8, _D), jnp.float32),
            pltpu.VMEM_SHARED((_NP, 16), jnp.float32),
        ],
    )
    def k(dst_hbm, out_hbm, dst_v, ones_v, zbuf, hbuf, vbuf, acc):
        c = lax.axis_index("c")
        s = lax.axis_index("s")
        w = c * _NS + s
        pltpu.sync_copy(dst_hbm.at[w], dst_v)
        for i in range(_CH):
            ones_v[i, :] = jnp.ones((16,), jnp.float32)
        for i in range(16):
            zbuf[i, :] = jnp.zeros((16,), jnp.float32)
        for q in range(40):
            pltpu.sync_copy(zbuf, acc.at[pl.ds(s * _RPT + 16 * q, 16)])
        plsc.subcore_barrier()

        for h in range(_NH):
            def body(j, carry):
                pltpu.sync_copy(ones_v, acc.at[dst_v.at[h, j]], add=True)
                return carry

            lax.fori_loop(0, _HCH, body, 0)
        plsc.subcore_barrier()
        for u in range(10):
            pltpu.sync_copy(acc.at[pl.ds(s * _RPT + 64 * u, 64)], hbuf)
            for r in range(8):
                for m in range(8):
                    vbuf[r, pl.ds(16 * m, 16)] = hbuf[8 * r + m, :]
            pltpu.sync_copy(vbuf, out_hbm.at[c, pl.ds(80 * s + 8 * u, 8)])

    return k(dst)


def _sc_gather_scatter(y, src, dst, zeros):
    @functools.partial(
        pl.kernel,
        mesh=_mesh(),
        out_type=jax.ShapeDtypeStruct((_NC * _NP, _D), jnp.float32),
        scratch_types=[
            pltpu.VMEM((_HCH, _CH), jnp.int32),
            pltpu.VMEM((_HCH, _CH), jnp.int32),
            pltpu.VMEM((_CH, _D), jnp.float32),
            pltpu.VMEM((_CH, _D), jnp.float32),
            pltpu.VMEM_SHARED((_NP, _D), jnp.float32),
            pltpu.SemaphoreType.DMA,
            pltpu.SemaphoreType.DMA,
            pltpu.SemaphoreType.DMA,
            pltpu.SemaphoreType.DMA,
        ],
    )
    def k(y_hbm, src_hbm, dst_hbm, zeros_hbm, out_hbm,
          src_v, dst_v, rows0_v, rows1_v, acc, gsem0, gsem1, ssem0, ssem1):
        c = lax.axis_index("c")
        s = lax.axis_index("s")
        w = c * _NS + s
        rows = (rows0_v, rows1_v)
        gsems = (gsem0, gsem1)
        ssems = (ssem0, ssem1)

        def gather(j, b):
            pltpu.async_copy(y_hbm.at[src_v.at[j]], rows[b], gsems[b])

        def gather_wait(j, b):
            pltpu.make_async_copy(y_hbm.at[src_v.at[j]], rows[b],
                                  gsems[b]).wait()

        def scatter(j, b):
            pltpu.async_copy(rows[b], acc.at[dst_v.at[j]], ssems[b], add=True)

        def scatter_wait(j, b):
            pltpu.make_async_copy(rows[b], acc.at[dst_v.at[j]],
                                  ssems[b]).wait()

        for h in range(_NH):
            pltpu.sync_copy(src_hbm.at[w, h], src_v)
            pltpu.sync_copy(dst_hbm.at[w, h], dst_v)
            gather(0, 0)
            if h == 0:
                pltpu.sync_copy(zeros_hbm.at[pl.ds(s * _RPT, _RPT)],
                                acc.at[pl.ds(s * _RPT, _RPT)])
                plsc.subcore_barrier()

            def body(t, carry):
                for b in (0, 1):
                    cur = 2 * t + b
                    gather_wait(cur, b)
                    scatter(cur, b)
                    ob = 1 - b

                    @pl.when(cur >= 1)
                    def _():
                        scatter_wait(cur - 1, ob)

                    @pl.when(cur + 1 < _HCH)
                    def _():
                        gather(cur + 1, ob)
                return carry

            lax.fori_loop(0, _HCH // 2, body, 0)
            scatter_wait(_HCH - 1, (_HCH - 1) % 2)
        plsc.subcore_barrier()
        pltpu.sync_copy(acc.at[pl.ds(s * _RPT, _RPT)],
                        out_hbm.at[pl.ds(c * _NP + s * _RPT, _RPT)])

    return k(y, src, dst, zeros)


def _tc_prologue(degp, x, w0):
    def body(degp_ref, x_ref, w_ref, y_ref, dinv_ref):
        deg = degp_ref[0:_N, 0:1] + degp_ref[0:_N, 1:2] + 1.0
        dinv = lax.rsqrt(deg)
        xw = jnp.dot(x_ref[...], w_ref[...], preferred_element_type=jnp.float32)
        y_ref[...] = xw * dinv
        dinv_ref[...] = dinv

    return pl.pallas_call(
        body,
        out_shape=[
            jax.ShapeDtypeStruct((_N, _D), jnp.float32),
            jax.ShapeDtypeStruct((_N, 1), jnp.float32),
        ],
    )(degp, x, w0)


def _bn_relu(p_ref, y_ref, dinv_ref, b_ref, g_ref, be_ref):
    conv = ((p_ref[0:_N, :] + p_ref[_NP:_NP + _N, :] + y_ref[...])
            * dinv_ref[...] + b_ref[...])
    m = jnp.mean(conv, axis=0, keepdims=True)
    v = jnp.mean((conv - m) ** 2, axis=0, keepdims=True)
    h = (conv - m) * lax.rsqrt(v + _EPS) * g_ref[...] + be_ref[...]
    return jnp.maximum(h, 0.0)


def _tc_layer(p, y, dinv, b, g, be, wn):
    def body(p_ref, y_ref, dinv_ref, b_ref, g_ref, be_ref, w_ref, o_ref):
        h = _bn_relu(p_ref, y_ref, dinv_ref, b_ref, g_ref, be_ref)
        o_ref[...] = (jnp.dot(h, w_ref[...], preferred_element_type=jnp.float32)
                      * dinv_ref[...])

    return pl.pallas_call(
        body,
        out_shape=jax.ShapeDtypeStruct((_N, _D), jnp.float32),
    )(p, y, dinv, b.reshape(1, _D), g.reshape(1, _D), be.reshape(1, _D), wn)


def _tc_head(p, y, dinv, b, g, be, batch, lw1, lb1, lw2, lb2):
    def body(p_ref, y_ref, dinv_ref, b_ref, g_ref, be_ref, batch_ref,
             lw1_ref, lb1_ref, lw2_ref, lb2_ref, o_ref):
        h = _bn_relu(p_ref, y_ref, dinv_ref, b_ref, g_ref, be_ref)
        gid = lax.broadcasted_iota(jnp.int32, (_N, _G), 1)
        onehot = (batch_ref[...] == gid).astype(jnp.float32)
        xsum = lax.dot_general(onehot, h, (((0,), (0,)), ((), ())),
                               preferred_element_type=jnp.float32)
        ones_col = jnp.ones((_N, 1), jnp.float32)
        counts = lax.dot_general(onehot, ones_col, (((0,), (0,)), ((), ())),
                                 preferred_element_type=jnp.float32)
        xmean = xsum / jnp.maximum(counts, 1.0)
        z = jnp.concatenate([xmean, xsum], axis=1)
        z = jnp.maximum(
            lax.dot_general(z, lw1_ref[...], (((1,), (1,)), ((), ())),
                            preferred_element_type=jnp.float32) + lb1_ref[...],
            0.0)
        o_ref[...] = lax.dot_general(z, lw2_ref[...], (((1,), (1,)), ((), ())),
                                     preferred_element_type=jnp.float32) + lb2_ref[...]

    return pl.pallas_call(
        body,
        out_shape=jax.ShapeDtypeStruct((_G, _O), jnp.float32),
    )(p, y, dinv, b.reshape(1, _D), g.reshape(1, _D), be.reshape(1, _D),
      batch, lw1, lb1.reshape(1, _D), lw2, lb2.reshape(1, _O))


def kernel(x, edge_index, batch, W0, b0, g0, be0, W1, b1, g1, be1,
           W2, b2, g2, be2, lw1, lb1, lw2, lb2):
    src = edge_index[0].astype(jnp.int32).reshape(_NW, _NH, _HCH, _CH)
    dst = edge_index[1].astype(jnp.int32).reshape(_NW, _NH, _HCH, _CH)
    batch2 = batch.astype(jnp.int32).reshape(_N, 1)
    zeros_nd = jnp.zeros((_NP, _D), jnp.float32)

    degr = _sc_degree(dst)
    degp = degr.reshape(_NC, _NS, _RPT, 16)[:, :, :, 0].reshape(_NC, _NP).T
    y, dinv = _tc_prologue(degp, x, W0)
    p = _sc_gather_scatter(y, src, dst, zeros_nd)
    y = _tc_layer(p, y, dinv, b0, g0, be0, W1)
    p = _sc_gather_scatter(y, src, dst, zeros_nd)
    y = _tc_layer(p, y, dinv, b1, g1, be1, W2)
    p = _sc_gather_scatter(y, src, dst, zeros_nd)
    return _tc_head(p, y, dinv, b2, g2, be2, batch2, lw1, lb1, lw2, lb2)


# speedup vs baseline: 1.0446x; 1.0446x over previous
import functools

import jax
import jax.numpy as jnp
from jax import lax
from jax.experimental import pallas as pl
from jax.experimental.pallas import tpu as pltpu
from jax.experimental.pallas import tpu_sc as plsc

_N = 10000
_E = 320000
_D = 128
_G = 128
_O = 10
_EPS = 1e-5

_NC = 2
_NS = 16
_NW = _NC * _NS
_EPT = _E // _NW
_CH = 100
_NCH = _EPT // _CH
_NH = 2
_HCH = _NCH // _NH
_NP = 10112
_RPT = _NP // _NS
_DEGW = _D


def _mesh():
    return plsc.VectorSubcoreMesh(core_axis_name="c", subcore_axis_name="s")


def _sc_degree(dst, ones, zeros):
    @functools.partial(
        pl.kernel,
        mesh=_mesh(),
        out_type=jax.ShapeDtypeStruct((_NC * _NP, _DEGW), jnp.float32),
        scratch_types=[
            pltpu.VMEM((_NH, _HCH, _CH), jnp.int32),
            pltpu.VMEM((_CH, _DEGW), jnp.float32),
            pltpu.VMEM_SHARED((_NP, _DEGW), jnp.float32),
        ],
    )
    def k(dst_hbm, ones_hbm, zeros_hbm, out_hbm, dst_v, ones_v, acc):
        c = lax.axis_index("c")
        s = lax.axis_index("s")
        w = c * _NS + s
        pltpu.sync_copy(zeros_hbm.at[pl.ds(s * _RPT, _RPT)],
                        acc.at[pl.ds(s * _RPT, _RPT)])
        pltpu.sync_copy(dst_hbm.at[w], dst_v)
        pltpu.sync_copy(ones_hbm, ones_v)
        plsc.subcore_barrier()

        for h in range(_NH):
            def body(j, carry):
                pltpu.sync_copy(ones_v, acc.at[dst_v.at[h, j]], add=True)
                return carry

            lax.fori_loop(0, _HCH, body, 0)
        plsc.subcore_barrier()
        pltpu.sync_copy(acc.at[pl.ds(s * _RPT, _RPT)],
                        out_hbm.at[pl.ds(c * _NP + s * _RPT, _RPT)])

    return k(dst, ones, zeros)


def _sc_gather_scatter(y, src, dst, zeros):
    @functools.partial(
        pl.kernel,
        mesh=_mesh(),
        out_type=jax.ShapeDtypeStruct((_NC * _NP, _D), jnp.float32),
        scratch_types=[
            pltpu.VMEM((_HCH, _CH), jnp.int32),
            pltpu.VMEM((_HCH, _CH), jnp.int32),
            pltpu.VMEM((_CH, _D), jnp.float32),
            pltpu.VMEM((_CH, _D), jnp.float32),
            pltpu.VMEM_SHARED((_NP, _D), jnp.float32),
            pltpu.SemaphoreType.DMA,
            pltpu.SemaphoreType.DMA,
            pltpu.SemaphoreType.DMA,
            pltpu.SemaphoreType.DMA,
        ],
    )
    def k(y_hbm, src_hbm, dst_hbm, zeros_hbm, out_hbm,
          src_v, dst_v, rows0_v, rows1_v, acc, gsem0, gsem1, ssem0, ssem1):
        c = lax.axis_index("c")
        s = lax.axis_index("s")
        w = c * _NS + s
        rows = (rows0_v, rows1_v)
        gsems = (gsem0, gsem1)
        ssems = (ssem0, ssem1)

        def gather(j, b):
            pltpu.async_copy(y_hbm.at[src_v.at[j]], rows[b], gsems[b])

        def gather_wait(j, b):
            pltpu.make_async_copy(y_hbm.at[src_v.at[j]], rows[b],
                                  gsems[b]).wait()

        def scatter(j, b):
            pltpu.async_copy(rows[b], acc.at[dst_v.at[j]], ssems[b], add=True)

        def scatter_wait(j, b):
            pltpu.make_async_copy(rows[b], acc.at[dst_v.at[j]],
                                  ssems[b]).wait()

        for h in range(_NH):
            pltpu.sync_copy(src_hbm.at[w, h], src_v)
            pltpu.sync_copy(dst_hbm.at[w, h], dst_v)
            gather(0, 0)
            if h == 0:
                pltpu.sync_copy(zeros_hbm.at[pl.ds(s * _RPT, _RPT)],
                                acc.at[pl.ds(s * _RPT, _RPT)])
                plsc.subcore_barrier()

            def body(t, carry):
                for b in (0, 1):
                    cur = 2 * t + b
                    gather_wait(cur, b)
                    scatter(cur, b)
                    ob = 1 - b

                    @pl.when(cur >= 1)
                    def _():
                        scatter_wait(cur - 1, ob)

                    @pl.when(cur + 1 < _HCH)
                    def _():
                        gather(cur + 1, ob)
                return carry

            lax.fori_loop(0, _HCH // 2, body, 0)
            scatter_wait(_HCH - 1, (_HCH - 1) % 2)
        plsc.subcore_barrier()
        pltpu.sync_copy(acc.at[pl.ds(s * _RPT, _RPT)],
                        out_hbm.at[pl.ds(c * _NP + s * _RPT, _RPT)])

    return k(y, src, dst, zeros)


def _tc_prologue(degp, x, w0):
    def body(degp_ref, x_ref, w_ref, y_ref, dinv_ref):
        deg = degp_ref[0:_N, 0:1] + degp_ref[_NP:_NP + _N, 0:1] + 1.0
        dinv = lax.rsqrt(deg)
        xw = jnp.dot(x_ref[...], w_ref[...], preferred_element_type=jnp.float32)
        y_ref[...] = xw * dinv
        dinv_ref[...] = dinv

    return pl.pallas_call(
        body,
        out_shape=[
            jax.ShapeDtypeStruct((_N, _D), jnp.float32),
            jax.ShapeDtypeStruct((_N, 1), jnp.float32),
        ],
    )(degp, x, w0)


def _bn_relu(p_ref, y_ref, dinv_ref, b_ref, g_ref, be_ref):
    conv = ((p_ref[0:_N, :] + p_ref[_NP:_NP + _N, :] + y_ref[...])
            * dinv_ref[...] + b_ref[...])
    m = jnp.mean(conv, axis=0, keepdims=True)
    v = jnp.mean((conv - m) ** 2, axis=0, keepdims=True)
    h = (conv - m) * lax.rsqrt(v + _EPS) * g_ref[...] + be_ref[...]
    return jnp.maximum(h, 0.0)


def _tc_layer(p, y, dinv, b, g, be, wn):
    def body(p_ref, y_ref, dinv_ref, b_ref, g_ref, be_ref, w_ref, o_ref):
        h = _bn_relu(p_ref, y_ref, dinv_ref, b_ref, g_ref, be_ref)
        o_ref[...] = (jnp.dot(h, w_ref[...], preferred_element_type=jnp.float32)
                      * dinv_ref[...])

    return pl.pallas_call(
        body,
        out_shape=jax.ShapeDtypeStruct((_N, _D), jnp.float32),
    )(p, y, dinv, b.reshape(1, _D), g.reshape(1, _D), be.reshape(1, _D), wn)


def _tc_head(p, y, dinv, b, g, be, batch, lw1, lb1, lw2, lb2):
    def body(p_ref, y_ref, dinv_ref, b_ref, g_ref, be_ref, batch_ref,
             lw1_ref, lb1_ref, lw2_ref, lb2_ref, o_ref):
        h = _bn_relu(p_ref, y_ref, dinv_ref, b_ref, g_ref, be_ref)
        gid = lax.broadcasted_iota(jnp.int32, (_N, _G), 1)
        onehot = (batch_ref[...] == gid).astype(jnp.float32)
        xsum = lax.dot_general(onehot, h, (((0,), (0,)), ((), ())),
                               preferred_element_type=jnp.float32)
        ones_col = jnp.ones((_N, 1), jnp.float32)
        counts = lax.dot_general(onehot, ones_col, (((0,), (0,)), ((), ())),
                                 preferred_element_type=jnp.float32)
        xmean = xsum / jnp.maximum(counts, 1.0)
        z = jnp.concatenate([xmean, xsum], axis=1)
        z = jnp.maximum(
            lax.dot_general(z, lw1_ref[...], (((1,), (1,)), ((), ())),
                            preferred_element_type=jnp.float32) + lb1_ref[...],
            0.0)
        o_ref[...] = lax.dot_general(z, lw2_ref[...], (((1,), (1,)), ((), ())),
                                     preferred_element_type=jnp.float32) + lb2_ref[...]

    return pl.pallas_call(
        body,
        out_shape=jax.ShapeDtypeStruct((_G, _O), jnp.float32),
    )(p, y, dinv, b.reshape(1, _D), g.reshape(1, _D), be.reshape(1, _D),
      batch, lw1, lb1.reshape(1, _D), lw2, lb2.reshape(1, _O))


def kernel(x, edge_index, batch, W0, b0, g0, be0, W1, b1, g1, be1,
           W2, b2, g2, be2, lw1, lb1, lw2, lb2):
    src = edge_index[0].astype(jnp.int32).reshape(_NW, _NH, _HCH, _CH)
    dst = edge_index[1].astype(jnp.int32).reshape(_NW, _NH, _HCH, _CH)
    batch2 = batch.astype(jnp.int32).reshape(_N, 1)
    zeros_nd = jnp.zeros((_NP, _D), jnp.float32)
    ones_ch = jnp.ones((_CH, _DEGW), jnp.float32)

    degp = _sc_degree(dst, ones_ch, zeros_nd)
    y, dinv = _tc_prologue(degp, x, W0)
    p = _sc_gather_scatter(y, src, dst, zeros_nd)
    y = _tc_layer(p, y, dinv, b0, g0, be0, W1)
    p = _sc_gather_scatter(y, src, dst, zeros_nd)
    y = _tc_layer(p, y, dinv, b1, g1, be1, W2)
    p = _sc_gather_scatter(y, src, dst, zeros_nd)
    return _tc_head(p, y, dinv, b2, g2, be2, batch2, lw1, lb1, lw2, lb2)


# speedup vs baseline: 1.1115x; 1.0641x over previous
import functools

import jax
import jax.numpy as jnp
from jax import lax
from jax.experimental import pallas as pl
from jax.experimental.pallas import tpu as pltpu
from jax.experimental.pallas import tpu_sc as plsc

_N = 10000
_E = 320000
_D = 128
_G = 128
_O = 10
_EPS = 1e-5

_NC = 2
_NS = 16
_NW = _NC * _NS
_EPT = _E // _NW
_CH = 125
_NCH = _EPT // _CH
_NH = 2
_HCH = _NCH // _NH
_NP = 10112
_RPT = _NP // _NS
_DEGW = _D


def _mesh():
    return plsc.VectorSubcoreMesh(core_axis_name="c", subcore_axis_name="s")


def _sc_degree(dst, ones, zeros):
    @functools.partial(
        pl.kernel,
        mesh=_mesh(),
        out_type=jax.ShapeDtypeStruct((_NC * _NP, _DEGW), jnp.float32),
        scratch_types=[
            pltpu.VMEM((_NH, _HCH, _CH), jnp.int32),
            pltpu.VMEM((_CH, _DEGW), jnp.float32),
            pltpu.VMEM_SHARED((_NP, _DEGW), jnp.float32),
        ],
    )
    def k(dst_hbm, ones_hbm, zeros_hbm, out_hbm, dst_v, ones_v, acc):
        c = lax.axis_index("c")
        s = lax.axis_index("s")
        w = c * _NS + s
        pltpu.sync_copy(zeros_hbm.at[pl.ds(s * _RPT, _RPT)],
                        acc.at[pl.ds(s * _RPT, _RPT)])
        pltpu.sync_copy(dst_hbm.at[w], dst_v)
        pltpu.sync_copy(ones_hbm, ones_v)
        plsc.subcore_barrier()

        for h in range(_NH):
            def body(j, carry):
                pltpu.sync_copy(ones_v, acc.at[dst_v.at[h, j]], add=True)
                return carry

            lax.fori_loop(0, _HCH, body, 0)
        plsc.subcore_barrier()
        pltpu.sync_copy(acc.at[pl.ds(s * _RPT, _RPT)],
                        out_hbm.at[pl.ds(c * _NP + s * _RPT, _RPT)])

    return k(dst, ones, zeros)


def _sc_gather_scatter(y, src, dst, zeros):
    @functools.partial(
        pl.kernel,
        mesh=_mesh(),
        out_type=jax.ShapeDtypeStruct((_NC * _NP, _D), jnp.float32),
        scratch_types=[
            pltpu.VMEM((_HCH, _CH), jnp.int32),
            pltpu.VMEM((_HCH, _CH), jnp.int32),
            pltpu.VMEM((_CH, _D), jnp.float32),
            pltpu.VMEM((_CH, _D), jnp.float32),
            pltpu.VMEM_SHARED((_NP, _D), jnp.float32),
            pltpu.SemaphoreType.DMA,
            pltpu.SemaphoreType.DMA,
            pltpu.SemaphoreType.DMA,
            pltpu.SemaphoreType.DMA,
        ],
    )
    def k(y_hbm, src_hbm, dst_hbm, zeros_hbm, out_hbm,
          src_v, dst_v, rows0_v, rows1_v, acc, gsem0, gsem1, ssem0, ssem1):
        c = lax.axis_index("c")
        s = lax.axis_index("s")
        w = c * _NS + s
        rows = (rows0_v, rows1_v)
        gsems = (gsem0, gsem1)
        ssems = (ssem0, ssem1)

        def gather(j, b):
            pltpu.async_copy(y_hbm.at[src_v.at[j]], rows[b], gsems[b])

        def gather_wait(j, b):
            pltpu.make_async_copy(y_hbm.at[src_v.at[j]], rows[b],
                                  gsems[b]).wait()

        def scatter(j, b):
            pltpu.async_copy(rows[b], acc.at[dst_v.at[j]], ssems[b], add=True)

        def scatter_wait(j, b):
            pltpu.make_async_copy(rows[b], acc.at[dst_v.at[j]],
                                  ssems[b]).wait()

        for h in range(_NH):
            pltpu.sync_copy(src_hbm.at[w, h], src_v)
            pltpu.sync_copy(dst_hbm.at[w, h], dst_v)
            gather(0, 0)
            if h == 0:
                pltpu.sync_copy(zeros_hbm.at[pl.ds(s * _RPT, _RPT)],
                                acc.at[pl.ds(s * _RPT, _RPT)])
                plsc.subcore_barrier()

            def body(t, carry):
                for b in (0, 1):
                    cur = 2 * t + b
                    gather_wait(cur, b)
                    scatter(cur, b)
                    ob = 1 - b

                    @pl.when(cur >= 1)
                    def _():
                        scatter_wait(cur - 1, ob)

                    @pl.when(cur + 1 < _HCH)
                    def _():
                        gather(cur + 1, ob)
                return carry

            lax.fori_loop(0, _HCH // 2, body, 0)
            scatter_wait(_HCH - 1, (_HCH - 1) % 2)
        plsc.subcore_barrier()
        pltpu.sync_copy(acc.at[pl.ds(s * _RPT, _RPT)],
                        out_hbm.at[pl.ds(c * _NP + s * _RPT, _RPT)])

    return k(y, src, dst, zeros)


def _tc_prologue(degp, x, w0):
    def body(degp_ref, x_ref, w_ref, y_ref, dinv_ref):
        deg = degp_ref[0:_N, 0:1] + degp_ref[_NP:_NP + _N, 0:1] + 1.0
        dinv = lax.rsqrt(deg)
        xw = jnp.dot(x_ref[...], w_ref[...], preferred_element_type=jnp.float32)
        y_ref[...] = xw * dinv
        dinv_ref[...] = dinv

    return pl.pallas_call(
        body,
        out_shape=[
            jax.ShapeDtypeStruct((_N, _D), jnp.float32),
            jax.ShapeDtypeStruct((_N, 1), jnp.float32),
        ],
    )(degp, x, w0)


def _bn_relu(p_ref, y_ref, dinv_ref, b_ref, g_ref, be_ref):
    conv = ((p_ref[0:_N, :] + p_ref[_NP:_NP + _N, :] + y_ref[...])
            * dinv_ref[...] + b_ref[...])
    m = jnp.mean(conv, axis=0, keepdims=True)
    v = jnp.mean((conv - m) ** 2, axis=0, keepdims=True)
    h = (conv - m) * lax.rsqrt(v + _EPS) * g_ref[...] + be_ref[...]
    return jnp.maximum(h, 0.0)


def _tc_layer(p, y, dinv, b, g, be, wn):
    def body(p_ref, y_ref, dinv_ref, b_ref, g_ref, be_ref, w_ref, o_ref):
        h = _bn_relu(p_ref, y_ref, dinv_ref, b_ref, g_ref, be_ref)
        o_ref[...] = (jnp.dot(h, w_ref[...], preferred_element_type=jnp.float32)
                      * dinv_ref[...])

    return pl.pallas_call(
        body,
        out_shape=jax.ShapeDtypeStruct((_N, _D), jnp.float32),
    )(p, y, dinv, b.reshape(1, _D), g.reshape(1, _D), be.reshape(1, _D), wn)


def _tc_head(p, y, dinv, b, g, be, batch, lw1, lb1, lw2, lb2):
    def body(p_ref, y_ref, dinv_ref, b_ref, g_ref, be_ref, batch_ref,
             lw1_ref, lb1_ref, lw2_ref, lb2_ref, o_ref):
        h = _bn_relu(p_ref, y_ref, dinv_ref, b_ref, g_ref, be_ref)
        gid = lax.broadcasted_iota(jnp.int32, (_N, _G), 1)
        onehot = (batch_ref[...] == gid).astype(jnp.float32)
        xsum = lax.dot_general(onehot, h, (((0,), (0,)), ((), ())),
                               preferred_element_type=jnp.float32)
        ones_col = jnp.ones((_N, 1), jnp.float32)
        counts = lax.dot_general(onehot, ones_col, (((0,), (0,)), ((), ())),
                                 preferred_element_type=jnp.float32)
        xmean = xsum / jnp.maximum(counts, 1.0)
        z = jnp.concatenate([xmean, xsum], axis=1)
        z = jnp.maximum(
            lax.dot_general(z, lw1_ref[...], (((1,), (1,)), ((), ())),
                            preferred_element_type=jnp.float32) + lb1_ref[...],
            0.0)
        o_ref[...] = lax.dot_general(z, lw2_ref[...], (((1,), (1,)), ((), ())),
                                     preferred_element_type=jnp.float32) + lb2_ref[...]

    return pl.pallas_call(
        body,
        out_shape=jax.ShapeDtypeStruct((_G, _O), jnp.float32),
    )(p, y, dinv, b.reshape(1, _D), g.reshape(1, _D), be.reshape(1, _D),
      batch, lw1, lb1.reshape(1, _D), lw2, lb2.reshape(1, _O))


def kernel(x, edge_index, batch, W0, b0, g0, be0, W1, b1, g1, be1,
           W2, b2, g2, be2, lw1, lb1, lw2, lb2):
    src = edge_index[0].astype(jnp.int32).reshape(_NW, _NH, _HCH, _CH)
    dst = edge_index[1].astype(jnp.int32).reshape(_NW, _NH, _HCH, _CH)
    batch2 = batch.astype(jnp.int32).reshape(_N, 1)
    zeros_nd = jnp.zeros((_NP, _D), jnp.float32)
    ones_ch = jnp.ones((_CH, _DEGW), jnp.float32)

    degp = _sc_degree(dst, ones_ch, zeros_nd)
    y, dinv = _tc_prologue(degp, x, W0)
    p = _sc_gather_scatter(y, src, dst, zeros_nd)
    y = _tc_layer(p, y, dinv, b0, g0, be0, W1)
    p = _sc_gather_scatter(y, src, dst, zeros_nd)
    y = _tc_layer(p, y, dinv, b1, g1, be1, W2)
    p = _sc_gather_scatter(y, src, dst, zeros_nd)
    return _tc_head(p, y, dinv, b2, g2, be2, batch2, lw1, lb1, lw2, lb2)
